# baseline (speedup 1.0000x reference)
import functools

import jax
import jax.numpy as jnp
from jax import lax
from jax.experimental import pallas as pl
from jax.experimental.pallas import tpu as pltpu
from jax.experimental.pallas import tpu_sc as plsc

N = 10000
H = 128
ESM = 1280
E = 160000
EL = 160000

NC, NS, L = 2, 16, 16
NW = NC * NS
CH = 624
CH_LAST = N - (NS - 1) * CH
EPT = E // NS
BLK = 80
NBLK = EPT // BLK
NB = 3
CHK = 25
NCHK = NBLK // CHK

CB = 128
NCB = EL // CB

_mesh = plsc.VectorSubcoreMesh(
    core_axis_name="c", subcore_axis_name="s", num_cores=NC, num_subcores=NS)


def _projA_body(xp_ref, wlp_ref, bp_ref, embp_ref, w1tp_ref, w1bp_ref,
                xr_ref, wlr_ref, br_ref, embr_ref, w1tr_ref, w1br_ref,
                yp_ref, yr_ref):
    pp = jnp.dot(xp_ref[...].astype(jnp.bfloat16),
                 wlp_ref[...].astype(jnp.bfloat16),
                 preferred_element_type=jnp.float32) + bp_ref[...]
    yp_ref[...] = (
        jnp.dot(pp, w1tp_ref[...], preferred_element_type=jnp.float32)
        + jnp.dot(embp_ref[...], w1bp_ref[...],
                  preferred_element_type=jnp.float32))
    pr = jnp.dot(xr_ref[...].astype(jnp.bfloat16),
                 wlr_ref[...].astype(jnp.bfloat16),
                 preferred_element_type=jnp.float32) + br_ref[...]
    yr_ref[...] = (
        jnp.dot(pr, w1tr_ref[...], preferred_element_type=jnp.float32)
        + jnp.dot(embr_ref[...], w1br_ref[...],
                  preferred_element_type=jnp.float32))


def _projA(xp, wlp, bp, embp, w1p, xr, wlr, br, embr, w1r):
    R = 1000
    xspec = pl.BlockSpec((R, ESM), lambda i: (i, 0))
    wspec = pl.BlockSpec((ESM, H), lambda i: (0, 0))
    bspec = pl.BlockSpec((1, H), lambda i: (0, 0))
    espec = pl.BlockSpec((R, H), lambda i: (i, 0))
    hspec = pl.BlockSpec((H, H), lambda i: (0, 0))
    yspec = pl.BlockSpec((R, H), lambda i: (i, 0))
    return pl.pallas_call(
        _projA_body,
        grid=(N // R,),
        in_specs=[xspec, wspec, bspec, espec, hspec, hspec,
                  xspec, wspec, bspec, espec, hspec, hspec],
        out_specs=[yspec, yspec],
        out_shape=[jax.ShapeDtypeStruct((N, H), jnp.float32),
                   jax.ShapeDtypeStruct((N, H), jnp.float32)],
    )(xp, wlp, bp.reshape(1, H), embp, w1p[:H], w1p[H:],
      xr, wlr, br.reshape(1, H), embr, w1r[:H], w1r[H:])


def _stageC_body(sp_ref, dp_ref, sr_ref, dr_ref, w2p_ref, w2r_ref,
                 y2p_ref, y2r_ref, ip_ref, ir_ref):
    inv_p = 1.0 / jnp.maximum(dp_ref[...], 1.0)
    inv_r = 1.0 / jnp.maximum(dr_ref[...], 1.0)
    hp = jnp.maximum(sp_ref[...] * inv_p, 0.0)
    hr = jnp.maximum(sr_ref[...] * inv_r, 0.0)
    y2p_ref[...] = jnp.dot(hp, w2p_ref[...], preferred_element_type=jnp.float32)
    y2r_ref[...] = jnp.dot(hr, w2r_ref[...], preferred_element_type=jnp.float32)
    ip_ref[...] = inv_p
    ir_ref[...] = inv_r


def _stageC(sum1_pep, deg_pep, sum1_prot, deg_prot, w2p, w2r):
    R = 1000
    return pl.pallas_call(
        _stageC_body,
        grid=(N // R,),
        in_specs=[
            pl.BlockSpec((R, H), lambda i: (i, 0)),
            pl.BlockSpec((R, 1), lambda i: (i, 0)),
            pl.BlockSpec((R, H), lambda i: (i, 0)),
            pl.BlockSpec((R, 1), lambda i: (i, 0)),
            pl.BlockSpec((H, H), lambda i: (0, 0)),
            pl.BlockSpec((H, H), lambda i: (0, 0)),
        ],
        out_specs=[
            pl.BlockSpec((R, H), lambda i: (i, 0)),
            pl.BlockSpec((R, H), lambda i: (i, 0)),
            pl.BlockSpec((R, 1), lambda i: (i, 0)),
            pl.BlockSpec((R, 1), lambda i: (i, 0)),
        ],
        out_shape=[
            jax.ShapeDtypeStruct((N, H), jnp.float32),
            jax.ShapeDtypeStruct((N, H), jnp.float32),
            jax.ShapeDtypeStruct((N, 1), jnp.float32),
            jax.ShapeDtypeStruct((N, 1), jnp.float32),
        ],
    )(sum1_pep, deg_pep.reshape(N, 1), sum1_prot, deg_prot.reshape(N, 1),
      w2p, w2r)


def _scale2_body(sp_ref, ip_ref, sr_ref, ir_ref, zp_ref, zr_ref):
    zp_ref[...] = sp_ref[...] * ip_ref[...]
    zr_ref[...] = sr_ref[...] * ir_ref[...]


def _scale2(sum2_pep, invd_pep, sum2_prot, invd_prot):
    R = 2000
    return pl.pallas_call(
        _scale2_body,
        grid=(N // R,),
        in_specs=[
            pl.BlockSpec((R, H), lambda i: (i, 0)),
            pl.BlockSpec((R, 1), lambda i: (i, 0)),
            pl.BlockSpec((R, H), lambda i: (i, 0)),
            pl.BlockSpec((R, 1), lambda i: (i, 0)),
        ],
        out_specs=[
            pl.BlockSpec((R, H), lambda i: (i, 0)),
            pl.BlockSpec((R, H), lambda i: (i, 0)),
        ],
        out_shape=[jax.ShapeDtypeStruct((N, H), jnp.float32),
                   jax.ShapeDtypeStruct((N, H), jnp.float32)],
    )(sum2_pep, invd_pep, sum2_prot, invd_prot)


def _make_segsum(with_deg):
    out_type = [
        jax.ShapeDtypeStruct((N, H), jnp.float32),
        jax.ShapeDtypeStruct((N, H), jnp.float32),
    ]
    if with_deg:
        out_type += [
            jax.ShapeDtypeStruct((N,), jnp.float32),
            jax.ShapeDtypeStruct((N,), jnp.float32),
        ]
    scratch = [
        pltpu.VMEM((CHK, 1, BLK), jnp.int32),
        pltpu.VMEM((CHK, 1, BLK), jnp.int32),
        pltpu.VMEM((NB, BLK, H), jnp.float32),
        pltpu.VMEM((BLK,), jnp.float32),
        pltpu.VMEM_SHARED((N, H), jnp.float32),
        pltpu.VMEM_SHARED((N,), jnp.float32),
        pltpu.SemaphoreType.DMA,
        pltpu.SemaphoreType.DMA,
        pltpu.SemaphoreType.DMA,
    ]

    def body(y_pep, y_prot, src_pp, dst_pp, src_rp, dst_rp, zrow, zdeg,
             *rest):
        if with_deg:
            sum_prot, sum_pep, deg_prot, deg_pep = rest[:4]
            scr = rest[4:]
        else:
            sum_prot, sum_pep = rest[:2]
            deg_prot = deg_pep = None
            scr = rest[2:]
        src_v, dst_v, rows_v, ones_v, acc_sh, dacc_sh, gsem, ssem, dsem = scr

        c = lax.axis_index("c")
        s = lax.axis_index("s")

        def run(y_hbm, src4, dst4, out_sum, out_deg):
            @pl.when(s < NS - 1)
            def _():
                pltpu.sync_copy(zrow.at[pl.ds(s * CH, CH)],
                                acc_sh.at[pl.ds(s * CH, CH)])

            @pl.when(s == NS - 1)
            def _():
                pltpu.sync_copy(zrow.at[pl.ds((NS - 1) * CH, CH_LAST)],
                                acc_sh.at[pl.ds((NS - 1) * CH, CH_LAST)])

            if with_deg:
                for i in range(BLK // L):
                    ones_v[pl.ds(i * L, L)] = jnp.full((L,), 1.0, jnp.float32)

                @pl.when(s == 0)
                def _():
                    pltpu.sync_copy(zdeg, dacc_sh)

            plsc.subcore_barrier()

            def chunk_body(ci, carry):
                pltpu.sync_copy(src4.at[s, ci], src_v)
                pltpu.sync_copy(dst4.at[s, ci], dst_v)
                for k in range(NB - 1):
                    pltpu.async_copy(
                        y_hbm.at[src_v.at[k, 0]], rows_v.at[k], gsem)

                def blk_body(b, carry2):
                    cur = lax.rem(b, NB)
                    dsl = dst_v.at[b, 0]
                    pltpu.make_async_copy(
                        y_hbm.at[src_v.at[b, 0]], rows_v.at[cur],
                        gsem).wait()
                    pltpu.async_copy(rows_v.at[cur], acc_sh.at[dsl], ssem,
                                     add=True)
                    if with_deg:
                        @pl.when(b > 0)
                        def _():
                            pltpu.make_async_copy(
                                ones_v, dacc_sh.at[dsl], dsem).wait()

                        pltpu.async_copy(ones_v, dacc_sh.at[dsl], dsem,
                                         add=True)

                    @pl.when(b + NB - 1 < CHK)
                    def _():
                        nxt = lax.rem(b + NB - 1, NB)

                        @pl.when(b >= 1)
                        def _():
                            pltpu.make_async_copy(
                                rows_v.at[cur], acc_sh.at[dsl], ssem).wait()

                        pltpu.async_copy(
                            y_hbm.at[src_v.at[b + NB - 1, 0]], rows_v.at[nxt],
                            gsem)

                    return carry2

                lax.fori_loop(0, CHK, blk_body, 0)
                for _ in range(NB):
                    pltpu.make_async_copy(
                        rows_v.at[0], acc_sh.at[dst_v.at[0, 0]], ssem).wait()
                if with_deg:
                    pltpu.make_async_copy(
                        ones_v, dacc_sh.at[dst_v.at[0, 0]], dsem).wait()
                return carry

            lax.fori_loop(0, NCHK, chunk_body, 0)

            plsc.subcore_barrier()

            @pl.when(s < NS - 1)
            def _():
                pltpu.sync_copy(acc_sh.at[pl.ds(s * CH, CH)],
                                out_sum.at[pl.ds(s * CH, CH)])

            @pl.when(s == NS - 1)
            def _():
                pltpu.sync_copy(acc_sh.at[pl.ds((NS - 1) * CH, CH_LAST)],
                                out_sum.at[pl.ds((NS - 1) * CH, CH_LAST)])

            if with_deg:
                @pl.when(s == 0)
                def _():
                    pltpu.sync_copy(dacc_sh, out_deg)

        @pl.when(c == 0)
        def _():
            run(y_pep, src_pp, dst_pp, sum_prot, deg_prot)

        @pl.when(c == 1)
        def _():
            run(y_prot, src_rp, dst_rp, sum_pep, deg_pep)

    return pl.kernel(body, out_type=out_type, mesh=_mesh,
                     scratch_types=scratch)


_segsum_deg = _make_segsum(True)
_segsum_nodeg = _make_segsum(False)


NT0 = NCB // NW
NTMAX = NT0 + 1


@functools.partial(
    pl.kernel,
    out_type=[
        jax.ShapeDtypeStruct((EL, H), jnp.float32),
        jax.ShapeDtypeStruct((EL, H), jnp.float32),
    ],
    mesh=_mesh,
    scratch_types=[
        pltpu.VMEM((NTMAX, 1, CB), jnp.int32),
        pltpu.VMEM((NTMAX, 1, CB), jnp.int32),
        pltpu.VMEM((3, CB, H), jnp.float32),
        pltpu.VMEM((3, CB, H), jnp.float32),
        pltpu.SemaphoreType.DMA,
        pltpu.SemaphoreType.DMA,
    ],
)
def _classifier_sc(zs_pep, zs_prot, eli_i, eli_j,
                   ag, bg,
                   iv, jv, av, bv, gsem, wsem):
    c = lax.axis_index("c")
    s = lax.axis_index("s")
    w = s * NC + c

    nt = jnp.where(w >= NW - 2, NTMAX, NT0)
    start = NT0 * w + jnp.maximum(w - (NW - 2), 0)

    pltpu.sync_copy(eli_i.at[pl.ds(start, NT0)], iv.at[pl.ds(0, NT0)])
    pltpu.sync_copy(eli_j.at[pl.ds(start, NT0)], jv.at[pl.ds(0, NT0)])

    @pl.when(nt == NTMAX)
    def _():
        pltpu.sync_copy(eli_i.at[pl.ds(start + NT0, 1)],
                        iv.at[pl.ds(NT0, 1)])
        pltpu.sync_copy(eli_j.at[pl.ds(start + NT0, 1)],
                        jv.at[pl.ds(NT0, 1)])

    def issue_gather(t):
        buf = lax.rem(t, 3)
        pltpu.async_copy(zs_pep.at[iv.at[t, 0]], av.at[buf], gsem)
        pltpu.async_copy(zs_prot.at[jv.at[t, 0]], bv.at[buf], gsem)

    issue_gather(0)
    issue_gather(1)

    def t_body(t, carry):
        @pl.when(t < nt)
        def _():
            buf = lax.rem(t, 3)
            off = (start + t) * CB
            pltpu.make_async_copy(
                zs_pep.at[iv.at[t, 0]], av.at[buf], gsem).wait()
            pltpu.make_async_copy(
                zs_prot.at[jv.at[t, 0]], bv.at[buf], gsem).wait()
            pltpu.async_copy(av.at[buf], ag.at[pl.ds(off, CB)], wsem)
            pltpu.async_copy(bv.at[buf], bg.at[pl.ds(off, CB)], wsem)

            @pl.when(t + 2 < nt)
            def _():
                @pl.when(t >= 1)
                def _():
                    pltpu.make_async_copy(
                        av.at[0], ag.at[pl.ds(0, CB)], wsem).wait()
                    pltpu.make_async_copy(
                        bv.at[0], bg.at[pl.ds(0, CB)], wsem).wait()

                issue_gather(t + 2)

        return carry

    lax.fori_loop(0, NTMAX, t_body, 0)

    for _ in range(3):
        pltpu.make_async_copy(av.at[0], ag.at[pl.ds(0, CB)], wsem).wait()
        pltpu.make_async_copy(bv.at[0], bg.at[pl.ds(0, CB)], wsem).wait()


def _finC_body(a_ref, b_ref, out_ref):
    ab = a_ref[...] * b_ref[...]
    out_ref[...] = jax.lax.dot(ab, jnp.ones((H,), jnp.float32),
                               preferred_element_type=jnp.float32)


def _finC(ag, bg):
    R = 256
    return pl.pallas_call(
        _finC_body,
        grid=(EL // R,),
        in_specs=[
            pl.BlockSpec((R, H), lambda i: (i, 0)),
            pl.BlockSpec((R, H), lambda i: (i, 0)),
        ],
        out_specs=pl.BlockSpec((R,), lambda i: (i,)),
        out_shape=jax.ShapeDtypeStruct((EL,), jnp.float32),
    )(ag, bg)


def kernel(pep_x, prot_x, pep_node_id, prot_node_id, edge_index_p2pr,
           edge_index_pr2p, edge_label_index, W_pep_lin, b_pep_lin,
           W_prot_lin, b_prot_lin, pep_emb, prot_emb, W1_p2pr, W1_pr2p,
           W2_p2pr, W2_pr2p):
    del pep_node_id, prot_node_id

    y1_pep, y1_prot = _projA(
        pep_x, W_pep_lin, b_pep_lin, pep_emb, W1_p2pr,
        prot_x, W_prot_lin, b_prot_lin, prot_emb, W1_pr2p)

    esh = (NS, NCHK, CHK, 1, BLK)
    src_pp = edge_index_p2pr[0].reshape(esh)
    dst_pp = edge_index_p2pr[1].reshape(esh)
    src_rp = edge_index_pr2p[0].reshape(esh)
    dst_rp = edge_index_pr2p[1].reshape(esh)
    zrow = jnp.zeros((N, H), jnp.float32)
    zdeg = jnp.zeros((N,), jnp.float32)

    sum1_prot, sum1_pep, deg_prot, deg_pep = _segsum_deg(
        y1_pep, y1_prot, src_pp, dst_pp, src_rp, dst_rp, zrow, zdeg)

    y2_pep, y2_prot, invd_pep, invd_prot = _stageC(
        sum1_pep, deg_pep, sum1_prot, deg_prot, W2_p2pr, W2_pr2p)

    sum2_prot, sum2_pep = _segsum_nodeg(
        y2_pep, y2_prot, src_pp, dst_pp, src_rp, dst_rp, zrow, zdeg)

    z2_pep, z2_prot = _scale2(sum2_pep, invd_pep, sum2_prot, invd_prot)

    ag, bg = _classifier_sc(
        z2_pep, z2_prot,
        edge_label_index[0].reshape(NCB, 1, CB),
        edge_label_index[1].reshape(NCB, 1, CB))
    return _finC(ag, bg)


# speedup vs baseline: 1.6542x; 1.6542x over previous
import functools

import jax
import jax.numpy as jnp
from jax import lax
from jax.experimental import pallas as pl
from jax.experimental.pallas import tpu as pltpu
from jax.experimental.pallas import tpu_sc as plsc

N = 10000
H = 128
ESM = 1280
E = 160000
EL = 160000

NC, NS, L = 2, 16, 16
NW = NC * NS
CH = 624
CH_LAST = N - (NS - 1) * CH
EPT = E // NS
BLK = 80
NBLK = EPT // BLK
NB = 3
CHK = 25
NCHK = NBLK // CHK

CB = 128
NCB = EL // CB

_mesh = plsc.VectorSubcoreMesh(
    core_axis_name="c", subcore_axis_name="s", num_cores=NC, num_subcores=NS)


def _projA_body(xp_ref, wlp_ref, bp_ref, embp_ref, w1tp_ref, w1bp_ref,
                xr_ref, wlr_ref, br_ref, embr_ref, w1tr_ref, w1br_ref,
                yp_ref, yr_ref):
    pp = jnp.dot(xp_ref[...].astype(jnp.bfloat16),
                 wlp_ref[...].astype(jnp.bfloat16),
                 preferred_element_type=jnp.float32) + bp_ref[...]
    yp_ref[...] = (
        jnp.dot(pp, w1tp_ref[...], preferred_element_type=jnp.float32)
        + jnp.dot(embp_ref[...], w1bp_ref[...],
                  preferred_element_type=jnp.float32))
    pr = jnp.dot(xr_ref[...].astype(jnp.bfloat16),
                 wlr_ref[...].astype(jnp.bfloat16),
                 preferred_element_type=jnp.float32) + br_ref[...]
    yr_ref[...] = (
        jnp.dot(pr, w1tr_ref[...], preferred_element_type=jnp.float32)
        + jnp.dot(embr_ref[...], w1br_ref[...],
                  preferred_element_type=jnp.float32))


def _projA(xp, wlp, bp, embp, w1p, xr, wlr, br, embr, w1r):
    R = 1000
    xspec = pl.BlockSpec((R, ESM), lambda i: (i, 0))
    wspec = pl.BlockSpec((ESM, H), lambda i: (0, 0))
    bspec = pl.BlockSpec((1, H), lambda i: (0, 0))
    espec = pl.BlockSpec((R, H), lambda i: (i, 0))
    hspec = pl.BlockSpec((H, H), lambda i: (0, 0))
    yspec = pl.BlockSpec((R, H), lambda i: (i, 0))
    return pl.pallas_call(
        _projA_body,
        grid=(N // R,),
        in_specs=[xspec, wspec, bspec, espec, hspec, hspec,
                  xspec, wspec, bspec, espec, hspec, hspec],
        out_specs=[yspec, yspec],
        out_shape=[jax.ShapeDtypeStruct((N, H), jnp.float32),
                   jax.ShapeDtypeStruct((N, H), jnp.float32)],
    )(xp, wlp, bp.reshape(1, H), embp, w1p[:H], w1p[H:],
      xr, wlr, br.reshape(1, H), embr, w1r[:H], w1r[H:])


def _stageC_body(sp_ref, dp_ref, sr_ref, dr_ref, w2p_ref, w2r_ref,
                 y2p_ref, y2r_ref, ip_ref, ir_ref):
    inv_p = 1.0 / jnp.maximum(dp_ref[...], 1.0)
    inv_r = 1.0 / jnp.maximum(dr_ref[...], 1.0)
    hp = jnp.maximum(sp_ref[...] * inv_p, 0.0)
    hr = jnp.maximum(sr_ref[...] * inv_r, 0.0)
    y2p_ref[...] = jnp.dot(hp, w2p_ref[...], preferred_element_type=jnp.float32)
    y2r_ref[...] = jnp.dot(hr, w2r_ref[...], preferred_element_type=jnp.float32)
    ip_ref[...] = inv_p
    ir_ref[...] = inv_r


def _stageC(sum1_pep, deg_pep, sum1_prot, deg_prot, w2p, w2r):
    R = 1000
    return pl.pallas_call(
        _stageC_body,
        grid=(N // R,),
        in_specs=[
            pl.BlockSpec((R, H), lambda i: (i, 0)),
            pl.BlockSpec((R, 1), lambda i: (i, 0)),
            pl.BlockSpec((R, H), lambda i: (i, 0)),
            pl.BlockSpec((R, 1), lambda i: (i, 0)),
            pl.BlockSpec((H, H), lambda i: (0, 0)),
            pl.BlockSpec((H, H), lambda i: (0, 0)),
        ],
        out_specs=[
            pl.BlockSpec((R, H), lambda i: (i, 0)),
            pl.BlockSpec((R, H), lambda i: (i, 0)),
            pl.BlockSpec((R, 1), lambda i: (i, 0)),
            pl.BlockSpec((R, 1), lambda i: (i, 0)),
        ],
        out_shape=[
            jax.ShapeDtypeStruct((N, H), jnp.float32),
            jax.ShapeDtypeStruct((N, H), jnp.float32),
            jax.ShapeDtypeStruct((N, 1), jnp.float32),
            jax.ShapeDtypeStruct((N, 1), jnp.float32),
        ],
    )(sum1_pep, deg_pep.reshape(N, 1), sum1_prot, deg_prot.reshape(N, 1),
      w2p, w2r)


def _scale2_body(sp_ref, ip_ref, sr_ref, ir_ref, zp_ref, zr_ref):
    zp_ref[...] = sp_ref[...] * ip_ref[...]
    zr_ref[...] = sr_ref[...] * ir_ref[...]


def _scale2(sum2_pep, invd_pep, sum2_prot, invd_prot):
    R = 2000
    return pl.pallas_call(
        _scale2_body,
        grid=(N // R,),
        in_specs=[
            pl.BlockSpec((R, H), lambda i: (i, 0)),
            pl.BlockSpec((R, 1), lambda i: (i, 0)),
            pl.BlockSpec((R, H), lambda i: (i, 0)),
            pl.BlockSpec((R, 1), lambda i: (i, 0)),
        ],
        out_specs=[
            pl.BlockSpec((R, H), lambda i: (i, 0)),
            pl.BlockSpec((R, H), lambda i: (i, 0)),
        ],
        out_shape=[jax.ShapeDtypeStruct((N, H), jnp.float32),
                   jax.ShapeDtypeStruct((N, H), jnp.float32)],
    )(sum2_pep, invd_pep, sum2_prot, invd_prot)


def _make_segsum(with_deg):
    out_type = [
        jax.ShapeDtypeStruct((N, H), jnp.float32),
        jax.ShapeDtypeStruct((N, H), jnp.float32),
    ]
    if with_deg:
        out_type += [
            jax.ShapeDtypeStruct((N,), jnp.float32),
            jax.ShapeDtypeStruct((N,), jnp.float32),
        ]
    scratch = [
        pltpu.VMEM((CHK, 1, BLK), jnp.int32),
        pltpu.VMEM((CHK, 1, BLK), jnp.int32),
        pltpu.VMEM((NB, BLK, H), jnp.float32),
        pltpu.VMEM((BLK,), jnp.float32),
        pltpu.VMEM_SHARED((N, H), jnp.float32),
        pltpu.VMEM_SHARED((N,), jnp.float32),
        pltpu.SemaphoreType.DMA,
        pltpu.SemaphoreType.DMA,
        pltpu.SemaphoreType.DMA,
    ]

    def body(y_pep, y_prot, src_pp, dst_pp, src_rp, dst_rp, zrow, zdeg,
             *rest):
        if with_deg:
            sum_prot, sum_pep, deg_prot, deg_pep = rest[:4]
            scr = rest[4:]
        else:
            sum_prot, sum_pep = rest[:2]
            deg_prot = deg_pep = None
            scr = rest[2:]
        src_v, dst_v, rows_v, ones_v, acc_sh, dacc_sh, gsem, ssem, dsem = scr

        c = lax.axis_index("c")
        s = lax.axis_index("s")

        def run(y_hbm, src4, dst4, out_sum, out_deg):
            @pl.when(s < NS - 1)
            def _():
                pltpu.sync_copy(zrow.at[pl.ds(s * CH, CH)],
                                acc_sh.at[pl.ds(s * CH, CH)])

            @pl.when(s == NS - 1)
            def _():
                pltpu.sync_copy(zrow.at[pl.ds((NS - 1) * CH, CH_LAST)],
                                acc_sh.at[pl.ds((NS - 1) * CH, CH_LAST)])

            if with_deg:
                for i in range(BLK // L):
                    ones_v[pl.ds(i * L, L)] = jnp.full((L,), 1.0, jnp.float32)

                @pl.when(s == 0)
                def _():
                    pltpu.sync_copy(zdeg, dacc_sh)

            plsc.subcore_barrier()

            def chunk_body(ci, carry):
                pltpu.sync_copy(src4.at[s, ci], src_v)
                pltpu.sync_copy(dst4.at[s, ci], dst_v)
                for k in range(NB - 1):
                    pltpu.async_copy(
                        y_hbm.at[src_v.at[k, 0]], rows_v.at[k], gsem)

                def blk_body(b, carry2):
                    cur = lax.rem(b, NB)
                    dsl = dst_v.at[b, 0]
                    pltpu.make_async_copy(
                        y_hbm.at[src_v.at[b, 0]], rows_v.at[cur],
                        gsem).wait()
                    pltpu.async_copy(rows_v.at[cur], acc_sh.at[dsl], ssem,
                                     add=True)
                    if with_deg:
                        @pl.when(b > 0)
                        def _():
                            pltpu.make_async_copy(
                                ones_v, dacc_sh.at[dsl], dsem).wait()

                        pltpu.async_copy(ones_v, dacc_sh.at[dsl], dsem,
                                         add=True)

                    @pl.when(b + NB - 1 < CHK)
                    def _():
                        nxt = lax.rem(b + NB - 1, NB)

                        @pl.when(b >= 1)
                        def _():
                            pltpu.make_async_copy(
                                rows_v.at[cur], acc_sh.at[dsl], ssem).wait()

                        pltpu.async_copy(
                            y_hbm.at[src_v.at[b + NB - 1, 0]], rows_v.at[nxt],
                            gsem)

                    return carry2

                lax.fori_loop(0, CHK, blk_body, 0)
                for _ in range(NB):
                    pltpu.make_async_copy(
                        rows_v.at[0], acc_sh.at[dst_v.at[0, 0]], ssem).wait()
                if with_deg:
                    pltpu.make_async_copy(
                        ones_v, dacc_sh.at[dst_v.at[0, 0]], dsem).wait()
                return carry

            lax.fori_loop(0, NCHK, chunk_body, 0)

            plsc.subcore_barrier()

            @pl.when(s < NS - 1)
            def _():
                pltpu.sync_copy(acc_sh.at[pl.ds(s * CH, CH)],
                                out_sum.at[pl.ds(s * CH, CH)])

            @pl.when(s == NS - 1)
            def _():
                pltpu.sync_copy(acc_sh.at[pl.ds((NS - 1) * CH, CH_LAST)],
                                out_sum.at[pl.ds((NS - 1) * CH, CH_LAST)])

            if with_deg:
                @pl.when(s == 0)
                def _():
                    pltpu.sync_copy(dacc_sh, out_deg)

        @pl.when(c == 0)
        def _():
            run(y_pep, src_pp, dst_pp, sum_prot, deg_prot)

        @pl.when(c == 1)
        def _():
            run(y_prot, src_rp, dst_rp, sum_pep, deg_pep)

    return pl.kernel(body, out_type=out_type, mesh=_mesh,
                     scratch_types=scratch)


_segsum_deg = _make_segsum(True)
_segsum_nodeg = _make_segsum(False)


NT0 = NCB // NW
NTMAX = NT0 + 1


@functools.partial(
    pl.kernel,
    out_type=[
        jax.ShapeDtypeStruct((EL, H), jnp.float32),
        jax.ShapeDtypeStruct((EL, H), jnp.float32),
    ],
    mesh=_mesh,
    scratch_types=[
        pltpu.VMEM((NTMAX, 1, CB), jnp.int32),
        pltpu.VMEM((NTMAX, 1, CB), jnp.int32),
        pltpu.VMEM((3, CB, H), jnp.float32),
        pltpu.VMEM((3, CB, H), jnp.float32),
        pltpu.SemaphoreType.DMA,
        pltpu.SemaphoreType.DMA,
    ],
)
def _classifier_sc(zs_pep, zs_prot, eli_i, eli_j,
                   ag, bg,
                   iv, jv, av, bv, gsem, wsem):
    c = lax.axis_index("c")
    s = lax.axis_index("s")
    w = s * NC + c

    nt = jnp.where(w >= NW - 2, NTMAX, NT0)
    start = NT0 * w + jnp.maximum(w - (NW - 2), 0)

    pltpu.sync_copy(eli_i.at[pl.ds(start, NT0)], iv.at[pl.ds(0, NT0)])
    pltpu.sync_copy(eli_j.at[pl.ds(start, NT0)], jv.at[pl.ds(0, NT0)])

    @pl.when(nt == NTMAX)
    def _():
        pltpu.sync_copy(eli_i.at[pl.ds(start + NT0, 1)],
                        iv.at[pl.ds(NT0, 1)])
        pltpu.sync_copy(eli_j.at[pl.ds(start + NT0, 1)],
                        jv.at[pl.ds(NT0, 1)])

    def issue_gather(t):
        buf = lax.rem(t, 3)
        pltpu.async_copy(zs_pep.at[iv.at[t, 0]], av.at[buf], gsem)
        pltpu.async_copy(zs_prot.at[jv.at[t, 0]], bv.at[buf], gsem)

    issue_gather(0)
    issue_gather(1)

    def t_body(t, carry):
        @pl.when(t < nt)
        def _():
            buf = lax.rem(t, 3)
            off = (start + t) * CB
            pltpu.make_async_copy(
                zs_pep.at[iv.at[t, 0]], av.at[buf], gsem).wait()
            pltpu.make_async_copy(
                zs_prot.at[jv.at[t, 0]], bv.at[buf], gsem).wait()
            pltpu.async_copy(av.at[buf], ag.at[pl.ds(off, CB)], wsem)
            pltpu.async_copy(bv.at[buf], bg.at[pl.ds(off, CB)], wsem)

            @pl.when(t + 2 < nt)
            def _():
                @pl.when(t >= 1)
                def _():
                    pltpu.make_async_copy(
                        av.at[0], ag.at[pl.ds(0, CB)], wsem).wait()
                    pltpu.make_async_copy(
                        bv.at[0], bg.at[pl.ds(0, CB)], wsem).wait()

                issue_gather(t + 2)

        return carry

    lax.fori_loop(0, NTMAX, t_body, 0)

    for _ in range(3):
        pltpu.make_async_copy(av.at[0], ag.at[pl.ds(0, CB)], wsem).wait()
        pltpu.make_async_copy(bv.at[0], bg.at[pl.ds(0, CB)], wsem).wait()


RFL = 1000
RFS = 8


def _finC_body(a_ref, b_ref, out_ref):
    ones = jnp.ones((1, H), jnp.float32)
    for k in range(RFS):
        ab = a_ref[pl.ds(k * RFL, RFL), :] * b_ref[pl.ds(k * RFL, RFL), :]
        out_ref[pl.ds(k, 1), :] = jax.lax.dot_general(
            ones, ab, dimension_numbers=(((1,), (1,)), ((), ())),
            preferred_element_type=jnp.float32)


def _finC(ag, bg):
    return pl.pallas_call(
        _finC_body,
        grid=(EL // (RFS * RFL),),
        in_specs=[
            pl.BlockSpec((RFS * RFL, H), lambda i: (i, 0)),
            pl.BlockSpec((RFS * RFL, H), lambda i: (i, 0)),
        ],
        out_specs=pl.BlockSpec((RFS, RFL), lambda i: (i, 0)),
        out_shape=jax.ShapeDtypeStruct((EL // RFL, RFL), jnp.float32),
    )(ag, bg)


def kernel(pep_x, prot_x, pep_node_id, prot_node_id, edge_index_p2pr,
           edge_index_pr2p, edge_label_index, W_pep_lin, b_pep_lin,
           W_prot_lin, b_prot_lin, pep_emb, prot_emb, W1_p2pr, W1_pr2p,
           W2_p2pr, W2_pr2p):
    del pep_node_id, prot_node_id

    y1_pep, y1_prot = _projA(
        pep_x, W_pep_lin, b_pep_lin, pep_emb, W1_p2pr,
        prot_x, W_prot_lin, b_prot_lin, prot_emb, W1_pr2p)

    esh = (NS, NCHK, CHK, 1, BLK)
    src_pp = edge_index_p2pr[0].reshape(esh)
    dst_pp = edge_index_p2pr[1].reshape(esh)
    src_rp = edge_index_pr2p[0].reshape(esh)
    dst_rp = edge_index_pr2p[1].reshape(esh)
    zrow = jnp.zeros((N, H), jnp.float32)
    zdeg = jnp.zeros((N,), jnp.float32)

    sum1_prot, sum1_pep, deg_prot, deg_pep = _segsum_deg(
        y1_pep, y1_prot, src_pp, dst_pp, src_rp, dst_rp, zrow, zdeg)

    y2_pep, y2_prot, invd_pep, invd_prot = _stageC(
        sum1_pep, deg_pep, sum1_prot, deg_prot, W2_p2pr, W2_pr2p)

    sum2_prot, sum2_pep = _segsum_nodeg(
        y2_pep, y2_prot, src_pp, dst_pp, src_rp, dst_rp, zrow, zdeg)

    z2_pep, z2_prot = _scale2(sum2_pep, invd_pep, sum2_prot, invd_prot)

    ag, bg = _classifier_sc(
        z2_pep, z2_prot,
        edge_label_index[0].reshape(NCB, 1, CB),
        edge_label_index[1].reshape(NCB, 1, CB))
    return _finC(ag, bg).reshape(EL)
